# baseline (speedup 1.0000x reference)
import functools

import jax
import jax.numpy as jnp
from jax import lax
from jax.experimental import pallas as pl
from jax.experimental.pallas import tpu as pltpu
from jax.experimental.pallas import tpu_sc as plsc

_N = 10000
_E = 160000
_F = 128
_T = 3
_R = 5
_H = 8
_DK = 16
_L = 2
_MAXT = 100

_NPAD = 10240
_NB = 512
_EPAD = 163840
_EB = 2048


def _adapt_body(x_ref, oh_ref, w_ref, b_ref, o_ref):
    x = x_ref[...]
    acc = jnp.zeros_like(x)
    for t in range(_T):
        m = oh_ref[:, t:t + 1]
        acc = acc + m * jnp.tanh(
            jnp.dot(x, w_ref[t], preferred_element_type=jnp.float32)
            + b_ref[t][None, :])
    o_ref[...] = acc


def _adapt(x, oh, w, b):
    return pl.pallas_call(
        _adapt_body,
        grid=(_NPAD // _NB,),
        in_specs=[
            pl.BlockSpec((_NB, _F), lambda i: (i, 0)),
            pl.BlockSpec((_NB, 8), lambda i: (i, 0)),
            pl.BlockSpec((_T, _F, _F), lambda i: (0, 0, 0)),
            pl.BlockSpec((_T, _F), lambda i: (0, 0)),
        ],
        out_specs=pl.BlockSpec((_NB, _F), lambda i: (i, 0)),
        out_shape=jax.ShapeDtypeStruct((_NPAD, _F), jnp.float32),
    )(x, oh, w, b)


def _node_body(h_ref, oh_ref, wk_ref, bk_ref, wq_ref, bq_ref, wv_ref, bv_ref,
               ba_ref, bm_ref, q_ref, kv_ref):
    h = h_ref[...]
    k = jnp.zeros_like(h)
    q = jnp.zeros_like(h)
    v = jnp.zeros_like(h)
    for t in range(_T):
        m = oh_ref[:, t:t + 1]
        k = k + m * (jnp.dot(h, wk_ref[t], preferred_element_type=jnp.float32)
                     + bk_ref[t][None, :])
        q = q + m * (jnp.dot(h, wq_ref[t], preferred_element_type=jnp.float32)
                     + bq_ref[t][None, :])
        v = v + m * (jnp.dot(h, wv_ref[t], preferred_element_type=jnp.float32)
                     + bv_ref[t][None, :])
    q_ref[...] = q
    for r in range(_R):
        kv_ref[:, r * 256:r * 256 + 128] = jnp.dot(
            k, ba_ref[r], preferred_element_type=jnp.float32)
        kv_ref[:, r * 256 + 128:r * 256 + 256] = jnp.dot(
            v, bm_ref[r], preferred_element_type=jnp.float32)


def _node(h, oh, wk, bk, wq, bq, wv, bv, blk_a, blk_m):
    return pl.pallas_call(
        _node_body,
        grid=(_NPAD // _NB,),
        in_specs=[
            pl.BlockSpec((_NB, _F), lambda i: (i, 0)),
            pl.BlockSpec((_NB, 8), lambda i: (i, 0)),
            pl.BlockSpec((_T, _F, _F), lambda i: (0, 0, 0)),
            pl.BlockSpec((_T, _F), lambda i: (0, 0)),
            pl.BlockSpec((_T, _F, _F), lambda i: (0, 0, 0)),
            pl.BlockSpec((_T, _F), lambda i: (0, 0)),
            pl.BlockSpec((_T, _F, _F), lambda i: (0, 0, 0)),
            pl.BlockSpec((_T, _F), lambda i: (0, 0)),
            pl.BlockSpec((_R, _F, _F), lambda i: (0, 0, 0)),
            pl.BlockSpec((_R, _F, _F), lambda i: (0, 0, 0)),
        ],
        out_specs=[
            pl.BlockSpec((_NB, _F), lambda i: (i, 0)),
            pl.BlockSpec((_NB, _R * 256), lambda i: (i, 0)),
        ],
        out_shape=[
            jax.ShapeDtypeStruct((_NPAD, _F), jnp.float32),
            jax.ShapeDtypeStruct((_NPAD, _R * 256), jnp.float32),
        ],
    )(h, oh, wk, bk, wq, bq, wv, bv, blk_a, blk_m)


def _spat_body(sk_ref, sv_ref, ba_ref, bm_ref, o_ref):
    sk = sk_ref[...]
    sv = sv_ref[...]
    for r in range(_R):
        o_ref[:, r * 256:r * 256 + 128] = jnp.dot(
            sk, ba_ref[r], preferred_element_type=jnp.float32)
        o_ref[:, r * 256 + 128:r * 256 + 256] = jnp.dot(
            sv, bm_ref[r], preferred_element_type=jnp.float32)


def _spat(sk, sv, blk_a, blk_m):
    return pl.pallas_call(
        _spat_body,
        grid=(1,),
        in_specs=[
            pl.BlockSpec((128, _F), lambda i: (0, 0)),
            pl.BlockSpec((128, _F), lambda i: (0, 0)),
            pl.BlockSpec((_R, _F, _F), lambda i: (0, 0, 0)),
            pl.BlockSpec((_R, _F, _F), lambda i: (0, 0, 0)),
        ],
        out_specs=pl.BlockSpec((128, _R * 256), lambda i: (0, 0)),
        out_shape=jax.ShapeDtypeStruct((128, _R * 256), jnp.float32),
    )(sk, sv, blk_a, blk_m)


def _att_body(gkv_ref, gs_ref, gq_ref, sb_ref, att_ref, vr_ref, gmax_ref):
    i = pl.program_id(0)
    kr = gkv_ref[:, :128] + gs_ref[:, :128]
    vr_ref[...] = gkv_ref[:, 128:] + gs_ref[:, 128:]
    att = jnp.dot(kr * gq_ref[...], sb_ref[...],
                  preferred_element_type=jnp.float32)
    att_ref[...] = att
    bmax = jnp.max(att, axis=0, keepdims=True)

    @pl.when(i == 0)
    def _():
        gmax_ref[...] = jnp.full((8, 16), -jnp.inf, jnp.float32)

    gmax_ref[...] = jnp.maximum(gmax_ref[...],
                                jnp.broadcast_to(bmax, (8, 16)))


def _att(gkv, gs, gq, sb):
    return pl.pallas_call(
        _att_body,
        grid=(_EPAD // _EB,),
        in_specs=[
            pl.BlockSpec((_EB, 256), lambda i: (i, 0)),
            pl.BlockSpec((_EB, 256), lambda i: (i, 0)),
            pl.BlockSpec((_EB, _F), lambda i: (i, 0)),
            pl.BlockSpec((_F, 16), lambda i: (0, 0)),
        ],
        out_specs=[
            pl.BlockSpec((_EB, 16), lambda i: (i, 0)),
            pl.BlockSpec((_EB, _F), lambda i: (i, 0)),
            pl.BlockSpec((8, 16), lambda i: (0, 0)),
        ],
        out_shape=[
            jax.ShapeDtypeStruct((_EPAD, 16), jnp.float32),
            jax.ShapeDtypeStruct((_EPAD, _F), jnp.float32),
            jax.ShapeDtypeStruct((8, 16), jnp.float32),
        ],
    )(gkv, gs, gq, sb)


def _msg_body(att_ref, vr_ref, gmax_ref, sbt_ref, ohm_ref, pm_ref,
              ecw_ref, msg_ref):
    g = gmax_ref[0:1, :]
    e = jnp.exp(att_ref[...] - g)
    ecw = jnp.zeros((att_ref.shape[0], _F), jnp.float32)
    for m in range(16):
        ecw = ecw + ohm_ref[:, m:m + 1] * jnp.dot(
            e, pm_ref[m], preferred_element_type=jnp.float32)
    ecw_ref[...] = ecw
    msg_ref[...] = vr_ref[...] * jnp.dot(e, sbt_ref[...],
                                         preferred_element_type=jnp.float32)


def _msg(att, vr, gmax, sbt, ohm, pm):
    return pl.pallas_call(
        _msg_body,
        grid=(_EPAD // _EB,),
        in_specs=[
            pl.BlockSpec((_EB, 16), lambda i: (i, 0)),
            pl.BlockSpec((_EB, _F), lambda i: (i, 0)),
            pl.BlockSpec((8, 16), lambda i: (0, 0)),
            pl.BlockSpec((16, _F), lambda i: (0, 0)),
            pl.BlockSpec((_EB, 16), lambda i: (i, 0)),
            pl.BlockSpec((16, 16, _F), lambda i: (0, 0, 0)),
        ],
        out_specs=[
            pl.BlockSpec((_EB, _F), lambda i: (i, 0)),
            pl.BlockSpec((_EB, _F), lambda i: (i, 0)),
        ],
        out_shape=[
            jax.ShapeDtypeStruct((_EPAD, _F), jnp.float32),
            jax.ShapeDtypeStruct((_EPAD, _F), jnp.float32),
        ],
    )(att, vr, gmax, sbt, ohm, pm)


def _comb_body(p_ref, dn_ref, h_ref, oh_ref, wa_ref, bba_ref, sig_ref,
               sbt_ref, o_ref):
    agg = p_ref[0] + p_ref[1]
    den = dn_ref[0] + dn_ref[1]
    recip = 1.0 / (den + 1e-16)
    aggn = agg * jnp.dot(recip, sbt_ref[...],
                         preferred_element_type=jnp.float32)
    h = h_ref[...]
    oh = oh_ref[...]
    trans = jnp.zeros_like(h)
    for t in range(_T):
        m = oh[:, t:t + 1]
        trans = trans + m * jax.nn.gelu(
            jnp.dot(aggn, wa_ref[t], preferred_element_type=jnp.float32)
            + bba_ref[t][None, :])
    alpha = jnp.sum(oh * sig_ref[...], axis=1, keepdims=True)
    o_ref[...] = trans * alpha + h * (1.0 - alpha)


def _comb(p, dn, h, oh, wa, ba, sig, sbt):
    return pl.pallas_call(
        _comb_body,
        grid=(_NPAD // _NB,),
        in_specs=[
            pl.BlockSpec((2, _NB, _F), lambda i: (0, i, 0)),
            pl.BlockSpec((2, _NB, 8), lambda i: (0, i, 0)),
            pl.BlockSpec((_NB, _F), lambda i: (i, 0)),
            pl.BlockSpec((_NB, 8), lambda i: (i, 0)),
            pl.BlockSpec((_T, _F, _F), lambda i: (0, 0, 0)),
            pl.BlockSpec((_T, _F), lambda i: (0, 0)),
            pl.BlockSpec((1, 8), lambda i: (0, 0)),
            pl.BlockSpec((8, _F), lambda i: (0, 0)),
        ],
        out_specs=pl.BlockSpec((_NB, _F), lambda i: (i, 0)),
        out_shape=jax.ShapeDtypeStruct((_NPAD, _F), jnp.float32),
    )(p, dn, h, oh, wa, ba, sig, sbt)


_NW = 32
_CH = 128
_CPW = _EPAD // (_NW * _CH)


_GCH = 64
_GCPW = _EPAD // (_NW * _GCH)


def _sc_gather(kv2, ss, q, ik3, it3, id3):
    mesh = plsc.VectorSubcoreMesh(core_axis_name="c", subcore_axis_name="s")

    @functools.partial(
        pl.kernel, mesh=mesh,
        out_type=[
            jax.ShapeDtypeStruct((_EPAD, 256), jnp.float32),
            jax.ShapeDtypeStruct((_EPAD, 256), jnp.float32),
            jax.ShapeDtypeStruct((_EPAD, _F), jnp.float32),
        ],
        scratch_types=[
            pltpu.VMEM((_GCPW, _GCH), jnp.int32),
            pltpu.VMEM((_GCPW, _GCH), jnp.int32),
            pltpu.VMEM((_GCPW, _GCH), jnp.int32),
            pltpu.VMEM((_GCH, 256), jnp.float32),
            pltpu.VMEM((_GCH, 256), jnp.float32),
            pltpu.VMEM((_GCH, 256), jnp.float32),
            pltpu.VMEM((_GCH, 256), jnp.float32),
            pltpu.VMEM((_GCH, _F), jnp.float32),
            pltpu.VMEM((_GCH, _F), jnp.float32),
            pltpu.SemaphoreType.DMA,
            pltpu.SemaphoreType.DMA,
            pltpu.SemaphoreType.DMA,
            pltpu.SemaphoreType.DMA,
            pltpu.SemaphoreType.DMA,
            pltpu.SemaphoreType.DMA,
        ],
    )
    def k(kv_hbm, ss_hbm, q_hbm, ik_hbm, it_hbm, id_hbm,
          gkv_hbm, gs_hbm, gq_hbm,
          ik_v, it_v, id_v, bkv0, bkv1, bs0, bs1, bq0, bq1,
          s10, s11, s20, s21, s30, s31):
        wid = lax.axis_index("s") * 2 + lax.axis_index("c")
        base = wid * (_GCPW * _GCH)
        pltpu.sync_copy(ik_hbm.at[wid], ik_v)
        pltpu.sync_copy(it_hbm.at[wid], it_v)
        pltpu.sync_copy(id_hbm.at[wid], id_v)
        bkv = (bkv0, bkv1)
        bs = (bs0, bs1)
        bq = (bq0, bq1)
        s1 = (s10, s11)
        s2 = (s20, s21)
        s3 = (s30, s31)

        def start(j, b):
            pltpu.async_copy(kv_hbm.at[ik_v.at[j]], bkv[b], s1[b])
            pltpu.async_copy(ss_hbm.at[it_v.at[j]], bs[b], s2[b])
            pltpu.async_copy(q_hbm.at[id_v.at[j]], bq[b], s3[b])

        def wait(j, b):
            pltpu.make_async_copy(kv_hbm.at[ik_v.at[j]], bkv[b],
                                  s1[b]).wait()
            pltpu.make_async_copy(ss_hbm.at[it_v.at[j]], bs[b],
                                  s2[b]).wait()
            pltpu.make_async_copy(q_hbm.at[id_v.at[j]], bq[b],
                                  s3[b]).wait()

        def out(j, b):
            pltpu.sync_copy(bkv[b], gkv_hbm.at[pl.ds(base + j * _GCH, _GCH)])
            pltpu.sync_copy(bs[b], gs_hbm.at[pl.ds(base + j * _GCH, _GCH)])
            pltpu.sync_copy(bq[b], gq_hbm.at[pl.ds(base + j * _GCH, _GCH)])

        start(0, 0)

        def body(g, carry):
            j0 = 2 * g
            j1 = j0 + 1
            wait(j0, 0)
            start(j1, 1)
            out(j0, 0)
            wait(j1, 1)

            @pl.when(j1 + 1 < _GCPW)
            def _():
                start(j1 + 1, 0)

            out(j1, 1)
            return carry

        lax.fori_loop(0, _GCPW // 2, body, 0)

    return k(kv2, ss, q, ik3, it3, id3)


_ND16 = _NPAD // 16


def _sc_scatter(msgu, ecw, id3, id83, zagg, zden8):
    mesh = plsc.VectorSubcoreMesh(core_axis_name="c", subcore_axis_name="s")

    @functools.partial(
        pl.kernel, mesh=mesh,
        out_type=[
            jax.ShapeDtypeStruct((2, _NPAD, _F), jnp.float32),
            jax.ShapeDtypeStruct((2, _ND16, _F), jnp.float32),
        ],
        scratch_types=[
            pltpu.VMEM((_CPW, _CH), jnp.int32),
            pltpu.VMEM((_CPW, _CH), jnp.int32),
            pltpu.VMEM((_CH, _F), jnp.float32),
            pltpu.VMEM((_CH, _F), jnp.float32),
            pltpu.VMEM_SHARED((_NPAD, _F), jnp.float32),
            pltpu.VMEM_SHARED((_ND16, _F), jnp.float32),
        ],
    )
    def k(msg_hbm, ecw_hbm, id_hbm, id8_hbm, zagg_hbm, zden_hbm,
          p_hbm, dn_hbm, id_v, id8_v, bm, bd, agg_sh, den_sh):
        cid = lax.axis_index("c")
        sid = lax.axis_index("s")
        wid = sid * 2 + cid
        base = wid * (_CPW * _CH)
        stripe = _NPAD // 16
        dstripe = _ND16 // 16

        def zbody(c, carry):
            row = sid * stripe + c * _CH
            pltpu.sync_copy(zagg_hbm.at[pl.ds(row, _CH)], bm)
            pltpu.sync_copy(bm, agg_sh.at[pl.ds(row, _CH)])
            return carry

        lax.fori_loop(0, stripe // _CH, zbody, 0)
        drow = sid * dstripe
        pltpu.sync_copy(zden_hbm.at[pl.ds(drow, dstripe)], bd.at[pl.ds(0, dstripe)])
        pltpu.sync_copy(bd.at[pl.ds(0, dstripe)], den_sh.at[pl.ds(drow, dstripe)])
        plsc.subcore_barrier()
        pltpu.sync_copy(id_hbm.at[wid], id_v)
        pltpu.sync_copy(id8_hbm.at[wid], id8_v)

        def body(j, carry):
            pltpu.sync_copy(msg_hbm.at[pl.ds(base + j * _CH, _CH)], bm)
            pltpu.sync_copy(bm, agg_sh.at[id_v.at[j]], add=True)
            pltpu.sync_copy(ecw_hbm.at[pl.ds(base + j * _CH, _CH)], bd)
            pltpu.sync_copy(bd, den_sh.at[id8_v.at[j]], add=True)
            return carry

        lax.fori_loop(0, _CPW, body, 0)
        plsc.subcore_barrier()

        def obody(c, carry):
            row = sid * stripe + c * _CH
            pltpu.sync_copy(agg_sh.at[pl.ds(row, _CH)], bm)
            pltpu.sync_copy(bm, p_hbm.at[cid, pl.ds(row, _CH)])
            return carry

        lax.fori_loop(0, stripe // _CH, obody, 0)
        pltpu.sync_copy(den_sh.at[pl.ds(drow, dstripe)], bd.at[pl.ds(0, dstripe)])
        pltpu.sync_copy(bd.at[pl.ds(0, dstripe)], dn_hbm.at[cid, pl.ds(drow, dstripe)])

    return k(msgu, ecw, id3, id83, zagg, zden8)


def _blockdiag(rel):
    ar = jnp.arange(_H)
    z = jnp.zeros((_R, _H, _DK, _H, _DK), rel.dtype)
    z = z.at[:, ar, :, ar, :].set(jnp.transpose(rel, (1, 0, 2, 3)))
    return z.reshape(_R, _H * _DK, _H * _DK)


def kernel(node_feature, node_type, edge_time, edge_index, edge_type,
           adapt_W, adapt_b, Wk, bk, Wq, bq, Wv, bv, Wa, ba,
           rel_att, rel_msg, rel_pri, skip, spat_k, spat_v):
    f32 = jnp.float32
    nt = node_type.astype(jnp.int32)
    src = edge_index[0].astype(jnp.int32)
    dst = edge_index[1].astype(jnp.int32)
    et = edge_type.astype(jnp.int32)
    tm = edge_time.astype(jnp.int32)

    oh = (nt[:, None] == jnp.arange(_T)[None, :]).astype(f32)
    oh = jnp.pad(oh, ((0, _NPAD - _N), (0, 8 - _T)))
    x = jnp.pad(node_feature.astype(f32), ((0, _NPAD - _N), (0, 0)))

    ik3 = jnp.pad(src * _R + et, (0, _EPAD - _E)).reshape(_NW, _GCPW, _GCH)
    it3 = jnp.pad(tm * _R + et, (0, _EPAD - _E)).reshape(_NW, _GCPW, _GCH)
    dst_p = jnp.pad(dst, (0, _EPAD - _E), constant_values=_N)
    id3g = dst_p.reshape(_NW, _GCPW, _GCH)
    id3 = dst_p.reshape(_NW, _CPW, _CH)
    id83 = (dst_p // 16).reshape(_NW, _CPW, _CH)
    ohm = (jnp.mod(dst_p, 16)[:, None] == jnp.arange(16)[None, :]).astype(f32)
    zagg = jnp.zeros((_NPAD, _F), f32)
    zden8 = jnp.zeros((_ND16, _F), f32)
    pm = jnp.zeros((16, 16, _F), f32)
    ar8 = jnp.arange(8)
    for m in range(16):
        pm = pm.at[m, ar8, m * 8 + ar8].set(1.0)

    ar128 = jnp.arange(128)
    sb = (ar128[:, None] // _DK == jnp.arange(16)[None, :]).astype(f32)
    sbt = sb.T
    sig = jnp.pad(jax.nn.sigmoid(skip.astype(f32)), ((0, 0), (0, 8 - _T)))

    h = _adapt(x, oh, adapt_W.astype(f32), adapt_b.astype(f32))

    scale = rel_pri.astype(f32)[:, :, :, None, None] / jnp.sqrt(
        jnp.asarray(_DK, f32))

    for l in range(_L):
        blk_a = _blockdiag(rel_att[l].astype(f32) * scale[l])
        blk_m = _blockdiag(rel_msg[l].astype(f32))

        q, kv = _node(h, oh, Wk[l].astype(f32), bk[l].astype(f32),
                      Wq[l].astype(f32), bq[l].astype(f32),
                      Wv[l].astype(f32), bv[l].astype(f32), blk_a, blk_m)
        kv2 = kv.reshape(_NPAD * _R, 256)

        sk_p = jnp.pad(spat_k[l].astype(f32), ((0, 128 - _MAXT), (0, 0)))
        sv_p = jnp.pad(spat_v[l].astype(f32), ((0, 128 - _MAXT), (0, 0)))
        ss = _spat(sk_p, sv_p, blk_a, blk_m).reshape(128 * _R, 256)

        gkv, gs, gq = _sc_gather(kv2, ss, q, ik3, it3, id3g)

        att, vr, gmax = _att(gkv, gs, gq, sb)
        ecw, msgu = _msg(att, vr, gmax, sbt, ohm, pm)

        p2, dn8 = _sc_scatter(msgu, ecw, id3, id83, zagg, zden8)
        dn2 = dn8.reshape(2, _NPAD, 8)

        h = _comb(p2, dn2, h, oh, Wa[l].astype(f32), ba[l].astype(f32),
                  sig[l:l + 1], sbt[:8])

    return h[:_N]


# speedup vs baseline: 1.1106x; 1.1106x over previous
import functools

import jax
import jax.numpy as jnp
from jax import lax
from jax.experimental import pallas as pl
from jax.experimental.pallas import tpu as pltpu
from jax.experimental.pallas import tpu_sc as plsc

_N = 10000
_E = 160000
_F = 128
_T = 3
_R = 5
_H = 8
_DK = 16
_L = 2
_MAXT = 100

_NPAD = 10240
_NB = 512
_EPAD = 163840
_EB = 2048


def _adapt_body(x_ref, oh_ref, w_ref, b_ref, o_ref):
    x = x_ref[...]
    acc = jnp.zeros_like(x)
    for t in range(_T):
        m = oh_ref[:, t:t + 1]
        acc = acc + m * jnp.tanh(
            jnp.dot(x, w_ref[t], preferred_element_type=jnp.float32)
            + b_ref[t][None, :])
    o_ref[...] = acc


def _adapt(x, oh, w, b):
    return pl.pallas_call(
        _adapt_body,
        grid=(_NPAD // _NB,),
        in_specs=[
            pl.BlockSpec((_NB, _F), lambda i: (i, 0)),
            pl.BlockSpec((_NB, 8), lambda i: (i, 0)),
            pl.BlockSpec((_T, _F, _F), lambda i: (0, 0, 0)),
            pl.BlockSpec((_T, _F), lambda i: (0, 0)),
        ],
        out_specs=pl.BlockSpec((_NB, _F), lambda i: (i, 0)),
        out_shape=jax.ShapeDtypeStruct((_NPAD, _F), jnp.float32),
    )(x, oh, w, b)


def _node_body(h_ref, oh_ref, wk_ref, bk_ref, wq_ref, bq_ref, wv_ref, bv_ref,
               ba_ref, bm_ref, q_ref, kv_ref):
    h = h_ref[...]
    k = jnp.zeros_like(h)
    q = jnp.zeros_like(h)
    v = jnp.zeros_like(h)
    for t in range(_T):
        m = oh_ref[:, t:t + 1]
        k = k + m * (jnp.dot(h, wk_ref[t], preferred_element_type=jnp.float32)
                     + bk_ref[t][None, :])
        q = q + m * (jnp.dot(h, wq_ref[t], preferred_element_type=jnp.float32)
                     + bq_ref[t][None, :])
        v = v + m * (jnp.dot(h, wv_ref[t], preferred_element_type=jnp.float32)
                     + bv_ref[t][None, :])
    q_ref[...] = q
    for r in range(_R):
        kv_ref[:, r * 256:r * 256 + 128] = jnp.dot(
            k, ba_ref[r], preferred_element_type=jnp.float32)
        kv_ref[:, r * 256 + 128:r * 256 + 256] = jnp.dot(
            v, bm_ref[r], preferred_element_type=jnp.float32)


def _node(h, oh, wk, bk, wq, bq, wv, bv, blk_a, blk_m):
    return pl.pallas_call(
        _node_body,
        grid=(_NPAD // _NB,),
        in_specs=[
            pl.BlockSpec((_NB, _F), lambda i: (i, 0)),
            pl.BlockSpec((_NB, 8), lambda i: (i, 0)),
            pl.BlockSpec((_T, _F, _F), lambda i: (0, 0, 0)),
            pl.BlockSpec((_T, _F), lambda i: (0, 0)),
            pl.BlockSpec((_T, _F, _F), lambda i: (0, 0, 0)),
            pl.BlockSpec((_T, _F), lambda i: (0, 0)),
            pl.BlockSpec((_T, _F, _F), lambda i: (0, 0, 0)),
            pl.BlockSpec((_T, _F), lambda i: (0, 0)),
            pl.BlockSpec((_R, _F, _F), lambda i: (0, 0, 0)),
            pl.BlockSpec((_R, _F, _F), lambda i: (0, 0, 0)),
        ],
        out_specs=[
            pl.BlockSpec((_NB, _F), lambda i: (i, 0)),
            pl.BlockSpec((_NB, _R * 256), lambda i: (i, 0)),
        ],
        out_shape=[
            jax.ShapeDtypeStruct((_NPAD, _F), jnp.float32),
            jax.ShapeDtypeStruct((_NPAD, _R * 256), jnp.float32),
        ],
    )(h, oh, wk, bk, wq, bq, wv, bv, blk_a, blk_m)


def _spat_body(sk_ref, sv_ref, ba_ref, bm_ref, o_ref):
    sk = sk_ref[...]
    sv = sv_ref[...]
    for r in range(_R):
        o_ref[:, r * 256:r * 256 + 128] = jnp.dot(
            sk, ba_ref[r], preferred_element_type=jnp.float32)
        o_ref[:, r * 256 + 128:r * 256 + 256] = jnp.dot(
            sv, bm_ref[r], preferred_element_type=jnp.float32)


def _spat(sk, sv, blk_a, blk_m):
    return pl.pallas_call(
        _spat_body,
        grid=(1,),
        in_specs=[
            pl.BlockSpec((128, _F), lambda i: (0, 0)),
            pl.BlockSpec((128, _F), lambda i: (0, 0)),
            pl.BlockSpec((_R, _F, _F), lambda i: (0, 0, 0)),
            pl.BlockSpec((_R, _F, _F), lambda i: (0, 0, 0)),
        ],
        out_specs=pl.BlockSpec((128, _R * 256), lambda i: (0, 0)),
        out_shape=jax.ShapeDtypeStruct((128, _R * 256), jnp.float32),
    )(sk, sv, blk_a, blk_m)


def _att_body(gkv_ref, gs_ref, gq_ref, sb_ref, sbt_ref, ohm_ref, pm_ref,
              ecw_ref, msg_ref):
    kr = gkv_ref[:, :128] + gs_ref[:, :128]
    vr = gkv_ref[:, 128:] + gs_ref[:, 128:]
    att = jnp.dot(kr * gq_ref[...], sb_ref[...],
                  preferred_element_type=jnp.float32)
    e = jnp.exp(att - 30.0)
    ecw = jnp.zeros((att.shape[0], _F), jnp.float32)
    for m in range(16):
        ecw = ecw + ohm_ref[:, m:m + 1] * jnp.dot(
            e, pm_ref[m], preferred_element_type=jnp.float32)
    ecw_ref[...] = ecw
    msg_ref[...] = vr * jnp.dot(e, sbt_ref[...],
                                preferred_element_type=jnp.float32)


def _att(gkv, gs, gq, sb, sbt, ohm, pm):
    return pl.pallas_call(
        _att_body,
        grid=(_EPAD // _EB,),
        in_specs=[
            pl.BlockSpec((_EB, 256), lambda i: (i, 0)),
            pl.BlockSpec((_EB, 256), lambda i: (i, 0)),
            pl.BlockSpec((_EB, _F), lambda i: (i, 0)),
            pl.BlockSpec((_F, 16), lambda i: (0, 0)),
            pl.BlockSpec((16, _F), lambda i: (0, 0)),
            pl.BlockSpec((_EB, 16), lambda i: (i, 0)),
            pl.BlockSpec((16, 16, _F), lambda i: (0, 0, 0)),
        ],
        out_specs=[
            pl.BlockSpec((_EB, _F), lambda i: (i, 0)),
            pl.BlockSpec((_EB, _F), lambda i: (i, 0)),
        ],
        out_shape=[
            jax.ShapeDtypeStruct((_EPAD, _F), jnp.float32),
            jax.ShapeDtypeStruct((_EPAD, _F), jnp.float32),
        ],
    )(gkv, gs, gq, sb, sbt, ohm, pm)


def _comb_body(p_ref, dn_ref, h_ref, oh_ref, wa_ref, bba_ref, sig_ref,
               sbt_ref, o_ref):
    agg = p_ref[0] + p_ref[1]
    den = dn_ref[0] + dn_ref[1]
    recip = 1.0 / (den + 1e-16)
    aggn = agg * jnp.dot(recip, sbt_ref[...],
                         preferred_element_type=jnp.float32)
    h = h_ref[...]
    oh = oh_ref[...]
    trans = jnp.zeros_like(h)
    for t in range(_T):
        m = oh[:, t:t + 1]
        trans = trans + m * jax.nn.gelu(
            jnp.dot(aggn, wa_ref[t], preferred_element_type=jnp.float32)
            + bba_ref[t][None, :])
    alpha = jnp.sum(oh * sig_ref[...], axis=1, keepdims=True)
    o_ref[...] = trans * alpha + h * (1.0 - alpha)


def _comb(p, dn, h, oh, wa, ba, sig, sbt):
    return pl.pallas_call(
        _comb_body,
        grid=(_NPAD // _NB,),
        in_specs=[
            pl.BlockSpec((2, _NB, _F), lambda i: (0, i, 0)),
            pl.BlockSpec((2, _NB, 8), lambda i: (0, i, 0)),
            pl.BlockSpec((_NB, _F), lambda i: (i, 0)),
            pl.BlockSpec((_NB, 8), lambda i: (i, 0)),
            pl.BlockSpec((_T, _F, _F), lambda i: (0, 0, 0)),
            pl.BlockSpec((_T, _F), lambda i: (0, 0)),
            pl.BlockSpec((1, 8), lambda i: (0, 0)),
            pl.BlockSpec((8, _F), lambda i: (0, 0)),
        ],
        out_specs=pl.BlockSpec((_NB, _F), lambda i: (i, 0)),
        out_shape=jax.ShapeDtypeStruct((_NPAD, _F), jnp.float32),
    )(p, dn, h, oh, wa, ba, sig, sbt)


_NW = 32
_CH = 128
_CPW = _EPAD // (_NW * _CH)
_GCH = 128
_GCPW = _EPAD // (_NW * _GCH)


def _sc_gather(kv2, ss, q, ik3, it3, id3):
    mesh = plsc.VectorSubcoreMesh(core_axis_name="c", subcore_axis_name="s")

    @functools.partial(
        pl.kernel, mesh=mesh,
        out_type=[
            jax.ShapeDtypeStruct((_EPAD, 256), jnp.float32),
            jax.ShapeDtypeStruct((_EPAD, 256), jnp.float32),
            jax.ShapeDtypeStruct((_EPAD, _F), jnp.float32),
        ],
        scratch_types=[
            pltpu.VMEM((_GCPW, _GCH), jnp.int32),
            pltpu.VMEM((_GCPW, _GCH), jnp.int32),
            pltpu.VMEM((_GCPW, _GCH), jnp.int32),
            pltpu.VMEM((_GCH, 256), jnp.float32),
            pltpu.VMEM((_GCH, 256), jnp.float32),
            pltpu.VMEM((_GCH, _F), jnp.float32),
            pltpu.SemaphoreType.DMA,
            pltpu.SemaphoreType.DMA,
            pltpu.SemaphoreType.DMA,
        ],
    )
    def k(kv_hbm, ss_hbm, q_hbm, ik_hbm, it_hbm, id_hbm,
          gkv_hbm, gs_hbm, gq_hbm,
          ik_v, it_v, id_v, bkv, bs, bq, s1, s2, s3):
        wid = lax.axis_index("s") * 2 + lax.axis_index("c")
        base = wid * (_GCPW * _GCH)
        pltpu.sync_copy(ik_hbm.at[wid], ik_v)
        pltpu.sync_copy(it_hbm.at[wid], it_v)
        pltpu.sync_copy(id_hbm.at[wid], id_v)

        def body(j, carry):
            c1 = pltpu.async_copy(kv_hbm.at[ik_v.at[j]], bkv, s1)
            c2 = pltpu.async_copy(ss_hbm.at[it_v.at[j]], bs, s2)
            c3 = pltpu.async_copy(q_hbm.at[id_v.at[j]], bq, s3)
            c1.wait()
            c2.wait()
            c3.wait()
            pltpu.sync_copy(bkv, gkv_hbm.at[pl.ds(base + j * _GCH, _GCH)])
            pltpu.sync_copy(bs, gs_hbm.at[pl.ds(base + j * _GCH, _GCH)])
            pltpu.sync_copy(bq, gq_hbm.at[pl.ds(base + j * _GCH, _GCH)])
            return carry

        lax.fori_loop(0, _GCPW, body, 0)

    return k(kv2, ss, q, ik3, it3, id3)


_ND16 = _NPAD // 16


def _sc_scatter(msgu, ecw, id3, id83, zagg, zden8):
    mesh = plsc.VectorSubcoreMesh(core_axis_name="c", subcore_axis_name="s")

    @functools.partial(
        pl.kernel, mesh=mesh,
        out_type=[
            jax.ShapeDtypeStruct((2, _NPAD, _F), jnp.float32),
            jax.ShapeDtypeStruct((2, _ND16, _F), jnp.float32),
        ],
        scratch_types=[
            pltpu.VMEM((_CPW, _CH), jnp.int32),
            pltpu.VMEM((_CPW, _CH), jnp.int32),
            pltpu.VMEM((_CH, _F), jnp.float32),
            pltpu.VMEM((_CH, _F), jnp.float32),
            pltpu.VMEM_SHARED((_NPAD, _F), jnp.float32),
            pltpu.VMEM_SHARED((_ND16, _F), jnp.float32),
        ],
    )
    def k(msg_hbm, ecw_hbm, id_hbm, id8_hbm, zagg_hbm, zden_hbm,
          p_hbm, dn_hbm, id_v, id8_v, bm, bd, agg_sh, den_sh):
        cid = lax.axis_index("c")
        sid = lax.axis_index("s")
        wid = sid * 2 + cid
        base = wid * (_CPW * _CH)
        stripe = _NPAD // 16
        dstripe = _ND16 // 16

        def zbody(c, carry):
            row = sid * stripe + c * _CH
            pltpu.sync_copy(zagg_hbm.at[pl.ds(row, _CH)], bm)
            pltpu.sync_copy(bm, agg_sh.at[pl.ds(row, _CH)])
            return carry

        lax.fori_loop(0, stripe // _CH, zbody, 0)
        drow = sid * dstripe
        pltpu.sync_copy(zden_hbm.at[pl.ds(drow, dstripe)], bd.at[pl.ds(0, dstripe)])
        pltpu.sync_copy(bd.at[pl.ds(0, dstripe)], den_sh.at[pl.ds(drow, dstripe)])
        plsc.subcore_barrier()
        pltpu.sync_copy(id_hbm.at[wid], id_v)
        pltpu.sync_copy(id8_hbm.at[wid], id8_v)

        def body(j, carry):
            pltpu.sync_copy(msg_hbm.at[pl.ds(base + j * _CH, _CH)], bm)
            pltpu.sync_copy(bm, agg_sh.at[id_v.at[j]], add=True)
            pltpu.sync_copy(ecw_hbm.at[pl.ds(base + j * _CH, _CH)], bd)
            pltpu.sync_copy(bd, den_sh.at[id8_v.at[j]], add=True)
            return carry

        lax.fori_loop(0, _CPW, body, 0)
        plsc.subcore_barrier()

        def obody(c, carry):
            row = sid * stripe + c * _CH
            pltpu.sync_copy(agg_sh.at[pl.ds(row, _CH)], bm)
            pltpu.sync_copy(bm, p_hbm.at[cid, pl.ds(row, _CH)])
            return carry

        lax.fori_loop(0, stripe // _CH, obody, 0)
        pltpu.sync_copy(den_sh.at[pl.ds(drow, dstripe)], bd.at[pl.ds(0, dstripe)])
        pltpu.sync_copy(bd.at[pl.ds(0, dstripe)], dn_hbm.at[cid, pl.ds(drow, dstripe)])

    return k(msgu, ecw, id3, id83, zagg, zden8)


def _blockdiag(rel):
    ar = jnp.arange(_H)
    z = jnp.zeros((_R, _H, _DK, _H, _DK), rel.dtype)
    z = z.at[:, ar, :, ar, :].set(jnp.transpose(rel, (1, 0, 2, 3)))
    return z.reshape(_R, _H * _DK, _H * _DK)


def kernel(node_feature, node_type, edge_time, edge_index, edge_type,
           adapt_W, adapt_b, Wk, bk, Wq, bq, Wv, bv, Wa, ba,
           rel_att, rel_msg, rel_pri, skip, spat_k, spat_v):
    f32 = jnp.float32
    nt = node_type.astype(jnp.int32)
    src = edge_index[0].astype(jnp.int32)
    dst = edge_index[1].astype(jnp.int32)
    et = edge_type.astype(jnp.int32)
    tm = edge_time.astype(jnp.int32)

    oh = (nt[:, None] == jnp.arange(_T)[None, :]).astype(f32)
    oh = jnp.pad(oh, ((0, _NPAD - _N), (0, 8 - _T)))
    x = jnp.pad(node_feature.astype(f32), ((0, _NPAD - _N), (0, 0)))

    ik3 = jnp.pad(src * _R + et, (0, _EPAD - _E)).reshape(_NW, _GCPW, _GCH)
    it3 = jnp.pad(tm * _R + et, (0, _EPAD - _E)).reshape(_NW, _GCPW, _GCH)
    dst_p = jnp.pad(dst, (0, _EPAD - _E), constant_values=_N)
    id3g = dst_p.reshape(_NW, _GCPW, _GCH)
    id3 = dst_p.reshape(_NW, _CPW, _CH)
    id83 = (dst_p // 16).reshape(_NW, _CPW, _CH)
    ohm = (jnp.mod(dst_p, 16)[:, None] == jnp.arange(16)[None, :]).astype(f32)
    zagg = jnp.zeros((_NPAD, _F), f32)
    zden8 = jnp.zeros((_ND16, _F), f32)
    pm = jnp.zeros((16, 16, _F), f32)
    ar8 = jnp.arange(8)
    for m in range(16):
        pm = pm.at[m, ar8, m * 8 + ar8].set(1.0)

    ar128 = jnp.arange(128)
    sb = (ar128[:, None] // _DK == jnp.arange(16)[None, :]).astype(f32)
    sbt = sb.T
    sig = jnp.pad(jax.nn.sigmoid(skip.astype(f32)), ((0, 0), (0, 8 - _T)))

    h = _adapt(x, oh, adapt_W.astype(f32), adapt_b.astype(f32))

    scale = rel_pri.astype(f32)[:, :, :, None, None] / jnp.sqrt(
        jnp.asarray(_DK, f32))

    for l in range(_L):
        blk_a = _blockdiag(rel_att[l].astype(f32) * scale[l])
        blk_m = _blockdiag(rel_msg[l].astype(f32))

        q, kv = _node(h, oh, Wk[l].astype(f32), bk[l].astype(f32),
                      Wq[l].astype(f32), bq[l].astype(f32),
                      Wv[l].astype(f32), bv[l].astype(f32), blk_a, blk_m)
        kv2 = kv.reshape(_NPAD * _R, 256)

        sk_p = jnp.pad(spat_k[l].astype(f32), ((0, 128 - _MAXT), (0, 0)))
        sv_p = jnp.pad(spat_v[l].astype(f32), ((0, 128 - _MAXT), (0, 0)))
        ss = _spat(sk_p, sv_p, blk_a, blk_m).reshape(128 * _R, 256)

        gkv, gs, gq = _sc_gather(kv2, ss, q, ik3, it3, id3g)

        ecw, msgu = _att(gkv, gs, gq, sb, sbt, ohm, pm)

        p2, dn8 = _sc_scatter(msgu, ecw, id3, id83, zagg, zden8)
        dn2 = dn8.reshape(2, _NPAD, 8)

        h = _comb(p2, dn2, h, oh, Wa[l].astype(f32), ba[l].astype(f32),
                  sig[l:l + 1], sbt[:8])

    return h[:_N]
